# baseline (speedup 1.0000x reference)
import functools

import jax
import jax.numpy as jnp
from jax import lax
from jax.experimental import pallas as pl
from jax.experimental.pallas import tpu as pltpu
from jax.experimental.pallas import tpu_sc as plsc

D = 64
L = 16
IDXW = 128
CHUNK = 512
NW = 32


def _lookup(idx2d, weight):
    nidx_rows = idx2d.shape[0]
    b_total = nidx_rows * IDXW
    per_w = b_total // NW
    nch = per_w // CHUNK
    g_per_chunk = CHUNK // IDXW
    idx_rows_per_w = per_w // IDXW

    mesh = plsc.VectorSubcoreMesh(core_axis_name="c", subcore_axis_name="s")

    @functools.partial(
        pl.kernel,
        out_type=jax.ShapeDtypeStruct((b_total, D), jnp.bfloat16),
        mesh=mesh,
        scratch_types=[
            pltpu.VMEM((g_per_chunk, IDXW), jnp.int32),
            pltpu.VMEM((CHUNK, D), jnp.float32),
            pltpu.VMEM((CHUNK, D), jnp.bfloat16),
            pltpu.SemaphoreType.DMA,
        ],
        compiler_params=pltpu.CompilerParams(
            needs_layout_passes=False, use_tc_tiling_on_sc=False
        ),
    )
    def run(idx_hbm, tbl_hbm, out_hbm, idx_v, rows_v, out_v, sem):
        cid = lax.axis_index("c")
        sid = lax.axis_index("s")
        wid = sid * 2 + cid
        idx_row0 = wid * idx_rows_per_w
        out_row0 = wid * per_w

        def chunk_body(t, carry):
            pltpu.sync_copy(
                idx_hbm.at[pl.ds(idx_row0 + t * g_per_chunk, g_per_chunk)],
                idx_v,
            )
            cps = []
            for g in range(g_per_chunk):
                cps.append(
                    pltpu.async_copy(
                        tbl_hbm.at[idx_v.at[g]],
                        rows_v.at[pl.ds(g * IDXW, IDXW)],
                        sem,
                    )
                )
            for cp in cps:
                cp.wait()

            iota = lax.iota(jnp.int32, L)

            def cast_row(j, c2):
                jv = jnp.full((L,), j, jnp.int32)
                for h in range(D // 32):
                    ev = plsc.load_gather(rows_v, [jv, h * 32 + 2 * iota])
                    od = plsc.load_gather(rows_v, [jv, h * 32 + 2 * iota + 1])
                    p = plsc.pack(ev, od, format=plsc.PackFormat.INTERLEAVED)
                    out_v[j, pl.ds(h * 32, 32)] = p
                return c2

            lax.fori_loop(0, CHUNK, cast_row, 0, unroll=2)
            pltpu.sync_copy(
                out_v, out_hbm.at[pl.ds(out_row0 + t * CHUNK, CHUNK)]
            )
            return carry

        lax.fori_loop(0, nch, chunk_body, 0)

    return run(idx2d, weight)


def kernel(input_ids, weight):
    b, s = input_ids.shape
    ids = input_ids.reshape(-1).astype(jnp.int32).reshape(-1, IDXW)
    out = _lookup(ids, weight)
    return out.reshape(b, s, D)
